# baseline (speedup 1.0000x reference)
import functools

import jax
import jax.numpy as jnp
from jax.experimental import pallas as pl
from jax.experimental.pallas import tpu as pltpu
from jax.experimental.pallas import tpu_sc as plsc

_COLS = 7


@functools.lru_cache(maxsize=1)
def _build_row_dot():
    mesh = plsc.ScalarSubcoreMesh(axis_name="c", num_cores=1)

    @functools.partial(
        pl.kernel,
        out_type=jax.ShapeDtypeStruct((8,), jnp.float32),
        mesh=mesh,
        scratch_types=[
            pltpu.SMEM((6 * _COLS,), jnp.float32),
            pltpu.SMEM((_COLS,), jnp.float32),
            pltpu.SMEM((1,), jnp.int32),
            pltpu.SMEM((8,), jnp.float32),
        ],
    )
    def _row_dot(m_hbm, t_hbm, s_hbm, out_hbm, m_s, t_s, s_s, o_s):
        pltpu.sync_copy(m_hbm, m_s)
        pltpu.sync_copy(t_hbm, t_s)
        pltpu.sync_copy(s_hbm, s_s)
        base = s_s[0] * _COLS
        acc = m_s[base] * t_s[0]
        for j in range(1, _COLS):
            acc += m_s[base + j] * t_s[j]
        for j in range(8):
            o_s[j] = acc
        pltpu.sync_copy(o_s, out_hbm)

    return _row_dot


def kernel(M, theta, state):
    m_flat = M.reshape(M.shape[0] * M.shape[1])
    s = jnp.asarray(state, jnp.int32).reshape(1)
    out = _build_row_dot()(m_flat, theta, s)
    return out[0]


# speedup vs baseline: 8.3522x; 8.3522x over previous
import jax
import jax.numpy as jnp
from jax.experimental import pallas as pl
from jax.experimental.pallas import tpu as pltpu


def _row_dot(s_ref, m_ref, t_ref, o_ref):
    i = s_ref[0]
    acc = m_ref[i, 0] * t_ref[0]
    for j in range(1, 7):
        acc += m_ref[i, j] * t_ref[j]
    o_ref[0] = acc


def kernel(M, theta, state):
    s = jnp.asarray(state, jnp.int32).reshape(1)
    out = pl.pallas_call(
        _row_dot,
        out_shape=jax.ShapeDtypeStruct((1,), jnp.float32),
        in_specs=[
            pl.BlockSpec(memory_space=pltpu.SMEM),
            pl.BlockSpec(memory_space=pltpu.SMEM),
            pl.BlockSpec(memory_space=pltpu.SMEM),
        ],
        out_specs=pl.BlockSpec(memory_space=pltpu.SMEM),
    )(s, M, theta)
    return out.reshape(())
